# baseline (speedup 1.0000x reference)
import functools

import jax
import jax.numpy as jnp
from jax import lax
from jax.experimental import pallas as pl
from jax.experimental.pallas import tpu as pltpu
from jax.experimental.pallas import tpu_sc as plsc


def _make_topk(B, M, N, TM, nsample, r2):
    r2 = float(r2)

    def kern(c_ref, n_ref, idx_ref):
        b = pl.program_id(0)
        c = c_ref[0]
        n = n_ref[0]
        cn = jnp.sum(c * c, axis=1, keepdims=True)
        nn = jnp.sum(n * n, axis=1)[None, :]
        cross = lax.dot_general(c, n, (((1,), (1,)), ((), ())),
                                preferred_element_type=jnp.float32)
        d2 = cn + nn - 2.0 * cross
        iota = lax.broadcasted_iota(jnp.int32, (TM, N), 1)
        iota_ones = jnp.concatenate(
            [lax.broadcasted_iota(jnp.int32, (N, 1), 0).astype(jnp.float32),
             jnp.ones((N, 1), jnp.float32)], axis=1)
        big = jnp.float32(3e38)
        cols = []
        a0 = None
        for s in range(nsample):
            v = jnp.min(d2, axis=1, keepdims=True)
            m = d2 <= v
            mf = m.astype(jnp.float32)
            af = lax.dot_general(mf, iota_ones, (((1,), (0,)), ((), ())),
                                 preferred_element_type=jnp.float32)
            amin = lax.cond(
                jnp.max(af[:, 1]) > 1.5,
                lambda: jnp.min(jnp.where(m, iota, N), axis=1),
                lambda: af[:, 0].astype(jnp.int32))
            if s == 0:
                a0 = amin
                chosen = amin
            else:
                chosen = jnp.where(v[:, 0] <= r2, amin, a0)
            cols.append(chosen[:, None])
            d2 = jnp.where(iota == amin[:, None], big, d2)
        idx_ref[0] = jnp.concatenate(cols, axis=1) + b * N

    return pl.pallas_call(
        kern,
        grid=(B, M // TM),
        in_specs=[pl.BlockSpec((1, TM, 3), lambda b, t: (b, t, 0)),
                  pl.BlockSpec((1, N, 3), lambda b, t: (b, 0, 0))],
        out_specs=pl.BlockSpec((1, TM, nsample), lambda b, t: (b, t, 0)),
        out_shape=jax.ShapeDtypeStruct((B, M, nsample), jnp.int32),
        compiler_params=pltpu.CompilerParams(
            dimension_semantics=("parallel", "parallel")),
    )


def _sc_gather(table, idx, D):
    total = idx.shape[0]
    info = plsc.get_sparse_core_info()
    nw = info.num_cores * info.num_subcores
    per_w = total // nw
    mesh = plsc.VectorSubcoreMesh(core_axis_name="c", subcore_axis_name="s")

    @functools.partial(
        pl.kernel, mesh=mesh,
        compiler_params=pltpu.CompilerParams(use_tc_tiling_on_sc=False),
        out_type=jax.ShapeDtypeStruct((total, D), jnp.float32),
        scratch_types=[pltpu.VMEM((per_w,), jnp.int32),
                       pltpu.VMEM((per_w, D), jnp.float32),
                       pltpu.SemaphoreType.DMA],
    )
    def k(table_hbm, idx_hbm, out_hbm, idx_v, rows_v, sem):
        wid = lax.axis_index("s") * info.num_cores + lax.axis_index("c")
        base = wid * per_w
        pltpu.sync_copy(idx_hbm.at[pl.ds(base, per_w)], idx_v)
        pltpu.async_copy(table_hbm.at[idx_v], rows_v, sem).wait()
        pltpu.sync_copy(rows_v, out_hbm.at[pl.ds(base, per_w)])

    return k(table, idx)


def _make_mlp(R, TM, D, F, H0, H1, S, DOUT=None):
    def kern(g_ref, c_ref, w0_ref, b0_ref, w1_ref, b1_ref, o_ref):
        g = g_ref[...]
        c = c_ref[...]
        crep = jnp.reshape(jnp.broadcast_to(c[:, None, :], (TM, S, 3)),
                           (TM * S, 3))
        x = jnp.concatenate([g[:, :3] - crep, g[:, 3:3 + F]], axis=1)
        h = jnp.dot(x, w0_ref[...], preferred_element_type=jnp.float32)
        h = jnp.maximum(h + b0_ref[...], 0.0)
        h = jnp.dot(h, w1_ref[...], preferred_element_type=jnp.float32)
        h = jnp.maximum(h + b1_ref[...], 0.0)
        acc = jnp.max(jnp.reshape(h, (TM, S, H1)), axis=1)
        if DOUT is None:
            o_ref[...] = acc
        else:
            pad = jnp.zeros((TM, DOUT - 3 - H1), jnp.float32)
            o_ref[...] = jnp.concatenate([c, acc, pad], axis=1)

    wout = H1 if DOUT is None else DOUT
    return pl.pallas_call(
        kern,
        grid=(R // TM,),
        in_specs=[pl.BlockSpec((TM * S, D), lambda t: (t, 0)),
                  pl.BlockSpec((TM, 3), lambda t: (t, 0)),
                  pl.BlockSpec((3 + F, H0), lambda t: (0, 0)),
                  pl.BlockSpec((1, H0), lambda t: (0, 0)),
                  pl.BlockSpec((H0, H1), lambda t: (0, 0)),
                  pl.BlockSpec((1, H1), lambda t: (0, 0))],
        out_specs=pl.BlockSpec((TM, wout), lambda t: (t, 0)),
        out_shape=jax.ShapeDtypeStruct((R, wout), jnp.float32),
        compiler_params=pltpu.CompilerParams(
            dimension_semantics=("parallel",)),
    )


def _make_mlp_vote(R, TM, D, F, H0, H1, S, DOUT):
    def kern(g_ref, c_ref, w0_ref, b0_ref, w1_ref, b1_ref,
             vw0_ref, vb0_ref, vwr_ref, vbr_ref,
             o_ref, off_ref, v_ref):
        g = g_ref[...]
        c = c_ref[...]
        crep = jnp.reshape(jnp.broadcast_to(c[:, None, :], (TM, S, 3)),
                           (TM * S, 3))
        x = jnp.concatenate([g[:, :3] - crep, g[:, 3:3 + F]], axis=1)
        h = jnp.dot(x, w0_ref[...], preferred_element_type=jnp.float32)
        h = jnp.maximum(h + b0_ref[...], 0.0)
        h = jnp.dot(h, w1_ref[...], preferred_element_type=jnp.float32)
        h = jnp.maximum(h + b1_ref[...], 0.0)
        acc = jnp.max(jnp.reshape(h, (TM, S, H1)), axis=1)
        pad = jnp.zeros((TM, DOUT - 3 - H1), jnp.float32)
        o_ref[...] = jnp.concatenate([c, acc, pad], axis=1)
        nf = jnp.dot(acc, vw0_ref[...], preferred_element_type=jnp.float32)
        nf = jnp.maximum(nf + vb0_ref[...], 0.0)
        off = jnp.dot(nf, vwr_ref[...], preferred_element_type=jnp.float32)
        off = off + vbr_ref[...]
        col = lax.broadcasted_iota(jnp.int32, (TM, 3), 1)
        mtr = jnp.where(col < 2, jnp.float32(3.0), jnp.float32(2.0))
        off_ref[...] = off
        v_ref[...] = c + jnp.clip(off, -mtr, mtr)

    return pl.pallas_call(
        kern,
        grid=(R // TM,),
        in_specs=[pl.BlockSpec((TM * S, D), lambda t: (t, 0)),
                  pl.BlockSpec((TM, 3), lambda t: (t, 0)),
                  pl.BlockSpec((3 + F, H0), lambda t: (0, 0)),
                  pl.BlockSpec((1, H0), lambda t: (0, 0)),
                  pl.BlockSpec((H0, H1), lambda t: (0, 0)),
                  pl.BlockSpec((1, H1), lambda t: (0, 0)),
                  pl.BlockSpec((H1, H1), lambda t: (0, 0)),
                  pl.BlockSpec((1, H1), lambda t: (0, 0)),
                  pl.BlockSpec((H1, 3), lambda t: (0, 0)),
                  pl.BlockSpec((1, 3), lambda t: (0, 0))],
        out_specs=[pl.BlockSpec((TM, DOUT), lambda t: (t, 0)),
                   pl.BlockSpec((TM, 3), lambda t: (t, 0)),
                   pl.BlockSpec((TM, 3), lambda t: (t, 0))],
        out_shape=[jax.ShapeDtypeStruct((R, DOUT), jnp.float32),
                   jax.ShapeDtypeStruct((R, 3), jnp.float32),
                   jax.ShapeDtypeStruct((R, 3), jnp.float32)],
        compiler_params=pltpu.CompilerParams(
            dimension_semantics=("parallel",)),
    )


def kernel(points, batch_size, sa0_w0, sa0_b0, sa0_w1, sa0_b1,
           sa1_w0, sa1_b0, sa1_w1, sa1_b1,
           vote_w0, vote_b0, vote_reg_w, vote_reg_b,
           sa3_w0, sa3_b0, sa3_w1, sa3_b1):
    B = 4
    N = points.shape[0] // B
    xyz = points[:, 1:4].reshape(B, N, 3)

    c0 = xyz[:, :1024]
    table0 = jnp.pad(points[:, 1:5], ((0, 0), (0, 12)))
    idx0 = _make_topk(B, 1024, N, 512, 16, 0.8 * 0.8)(c0, xyz)
    g0 = _sc_gather(table0, idx0.reshape(-1), 16)
    table1 = _make_mlp(B * 1024, 1024, 16, 1, 16, 32, 16, DOUT=48)(
        g0, c0.reshape(-1, 3),
        sa0_w0, sa0_b0.reshape(1, -1), sa0_w1, sa0_b1.reshape(1, -1))

    c1 = c0[:, :256]
    idx1 = _make_topk(B, 256, 1024, 256, 16, 1.6 * 1.6)(c1, c0)
    g1 = _sc_gather(table1, idx1.reshape(-1), 48)
    table3, ctr_off, vote_xyz = _make_mlp_vote(
        B * 256, 512, 48, 32, 64, 128, 16, DOUT=144)(
        g1, c1.reshape(-1, 3),
        sa1_w0, sa1_b0.reshape(1, -1), sa1_w1, sa1_b1.reshape(1, -1),
        vote_w0, vote_b0.reshape(1, -1), vote_reg_w, vote_reg_b.reshape(1, -1))

    idx3 = _make_topk(B, 256, 256, 256, 16, 4.8 * 4.8)(
        vote_xyz.reshape(B, 256, 3), c1)
    g3 = _sc_gather(table3, idx3.reshape(-1), 144)
    f3 = _make_mlp(B * 256, 512, 144, 128, 256, 256, 16)(
        g3, vote_xyz,
        sa3_w0, sa3_b0.reshape(1, -1), sa3_w1, sa3_b1.reshape(1, -1))

    bz = (jnp.asarray(batch_size, jnp.int32) - jnp.int32(B)).astype(jnp.float32)
    ctr_batch = points[:, 0].reshape(B, N)[:, :256].reshape(-1) + bz
    centers = jnp.concatenate([ctr_batch[:, None], vote_xyz], axis=1)
    centers_origin = jnp.concatenate([ctr_batch[:, None], c1.reshape(-1, 3)],
                                     axis=1)
    ctr_offsets = jnp.concatenate([ctr_batch[:, None], ctr_off], axis=1)
    return f3, centers, centers_origin, ctr_offsets


# speedup vs baseline: 1.9702x; 1.9702x over previous
import functools

import jax
import jax.numpy as jnp
from jax import lax
from jax.experimental import pallas as pl
from jax.experimental.pallas import tpu as pltpu
from jax.experimental.pallas import tpu_sc as plsc


def _make_topk(B, M, N, TM, nsample, r2):
    r2 = float(r2)

    def kern(c_ref, n_ref, idx_ref):
        b = pl.program_id(0)
        c = c_ref[0]
        n = n_ref[0]
        cn = jnp.sum(c * c, axis=1, keepdims=True)
        nn = jnp.sum(n * n, axis=1)[None, :]
        cross = lax.dot_general(c, n, (((1,), (1,)), ((), ())),
                                preferred_element_type=jnp.float32)
        d2 = cn + nn - 2.0 * cross
        iotaf = lax.broadcasted_iota(
            jnp.int32, (TM, N), 1).astype(jnp.float32)
        big = jnp.float32(3e38)
        cols = []
        a0 = None
        for s in range(nsample):
            v = jnp.min(d2, axis=1, keepdims=True)
            m = d2 <= v
            amin = jnp.min(jnp.where(m, iotaf, big), axis=1)
            if s == 0:
                a0 = amin
                chosen = amin
            else:
                chosen = jnp.where(v[:, 0] <= r2, amin, a0)
            cols.append(chosen[:, None])
            d2 = jnp.where(iotaf == amin[:, None], big, d2)
        idx = jnp.concatenate(cols, axis=1).astype(jnp.int32)
        idx_ref[0] = idx + b * N

    return pl.pallas_call(
        kern,
        grid=(B, M // TM),
        in_specs=[pl.BlockSpec((1, TM, 3), lambda b, t: (b, t, 0)),
                  pl.BlockSpec((1, N, 3), lambda b, t: (b, 0, 0))],
        out_specs=pl.BlockSpec((1, TM, nsample), lambda b, t: (b, t, 0)),
        out_shape=jax.ShapeDtypeStruct((B, M, nsample), jnp.int32),
        compiler_params=pltpu.CompilerParams(
            dimension_semantics=("parallel", "parallel")),
    )


def _sc_gather(table, idx, D):
    total = idx.shape[0]
    info = plsc.get_sparse_core_info()
    nw = info.num_cores * info.num_subcores
    per_w = total // nw
    mesh = plsc.VectorSubcoreMesh(core_axis_name="c", subcore_axis_name="s")

    @functools.partial(
        pl.kernel, mesh=mesh,
        compiler_params=pltpu.CompilerParams(use_tc_tiling_on_sc=False),
        out_type=jax.ShapeDtypeStruct((total, D), jnp.float32),
        scratch_types=[pltpu.VMEM((per_w,), jnp.int32),
                       pltpu.VMEM((per_w, D), jnp.float32),
                       pltpu.SemaphoreType.DMA],
    )
    def k(table_hbm, idx_hbm, out_hbm, idx_v, rows_v, sem):
        wid = lax.axis_index("s") * info.num_cores + lax.axis_index("c")
        base = wid * per_w
        pltpu.sync_copy(idx_hbm.at[pl.ds(base, per_w)], idx_v)
        pltpu.async_copy(table_hbm.at[idx_v], rows_v, sem).wait()
        pltpu.sync_copy(rows_v, out_hbm.at[pl.ds(base, per_w)])

    return k(table, idx)


def _make_mlp(R, TM, D, F, H0, H1, S, DOUT=None):
    def kern(g_ref, c_ref, w0_ref, b0_ref, w1_ref, b1_ref, o_ref):
        g = g_ref[...]
        c = c_ref[...]
        crep = jnp.reshape(jnp.broadcast_to(c[:, None, :], (TM, S, 3)),
                           (TM * S, 3))
        x = jnp.concatenate([g[:, :3] - crep, g[:, 3:3 + F]], axis=1)
        h = jnp.dot(x, w0_ref[...], preferred_element_type=jnp.float32)
        h = jnp.maximum(h + b0_ref[...], 0.0)
        h = jnp.dot(h, w1_ref[...], preferred_element_type=jnp.float32)
        h = jnp.maximum(h + b1_ref[...], 0.0)
        acc = jnp.max(jnp.reshape(h, (TM, S, H1)), axis=1)
        if DOUT is None:
            o_ref[...] = acc
        else:
            pad = jnp.zeros((TM, DOUT - 3 - H1), jnp.float32)
            o_ref[...] = jnp.concatenate([c, acc, pad], axis=1)

    wout = H1 if DOUT is None else DOUT
    return pl.pallas_call(
        kern,
        grid=(R // TM,),
        in_specs=[pl.BlockSpec((TM * S, D), lambda t: (t, 0)),
                  pl.BlockSpec((TM, 3), lambda t: (t, 0)),
                  pl.BlockSpec((3 + F, H0), lambda t: (0, 0)),
                  pl.BlockSpec((1, H0), lambda t: (0, 0)),
                  pl.BlockSpec((H0, H1), lambda t: (0, 0)),
                  pl.BlockSpec((1, H1), lambda t: (0, 0))],
        out_specs=pl.BlockSpec((TM, wout), lambda t: (t, 0)),
        out_shape=jax.ShapeDtypeStruct((R, wout), jnp.float32),
        compiler_params=pltpu.CompilerParams(
            dimension_semantics=("parallel",)),
    )


def _make_mlp_vote(R, TM, D, F, H0, H1, S, DOUT):
    def kern(g_ref, c_ref, w0_ref, b0_ref, w1_ref, b1_ref,
             vw0_ref, vb0_ref, vwr_ref, vbr_ref,
             o_ref, off_ref, v_ref):
        g = g_ref[...]
        c = c_ref[...]
        crep = jnp.reshape(jnp.broadcast_to(c[:, None, :], (TM, S, 3)),
                           (TM * S, 3))
        x = jnp.concatenate([g[:, :3] - crep, g[:, 3:3 + F]], axis=1)
        h = jnp.dot(x, w0_ref[...], preferred_element_type=jnp.float32)
        h = jnp.maximum(h + b0_ref[...], 0.0)
        h = jnp.dot(h, w1_ref[...], preferred_element_type=jnp.float32)
        h = jnp.maximum(h + b1_ref[...], 0.0)
        acc = jnp.max(jnp.reshape(h, (TM, S, H1)), axis=1)
        pad = jnp.zeros((TM, DOUT - 3 - H1), jnp.float32)
        o_ref[...] = jnp.concatenate([c, acc, pad], axis=1)
        nf = jnp.dot(acc, vw0_ref[...], preferred_element_type=jnp.float32)
        nf = jnp.maximum(nf + vb0_ref[...], 0.0)
        off = jnp.dot(nf, vwr_ref[...], preferred_element_type=jnp.float32)
        off = off + vbr_ref[...]
        col = lax.broadcasted_iota(jnp.int32, (TM, 3), 1)
        mtr = jnp.where(col < 2, jnp.float32(3.0), jnp.float32(2.0))
        off_ref[...] = off
        v_ref[...] = c + jnp.clip(off, -mtr, mtr)

    return pl.pallas_call(
        kern,
        grid=(R // TM,),
        in_specs=[pl.BlockSpec((TM * S, D), lambda t: (t, 0)),
                  pl.BlockSpec((TM, 3), lambda t: (t, 0)),
                  pl.BlockSpec((3 + F, H0), lambda t: (0, 0)),
                  pl.BlockSpec((1, H0), lambda t: (0, 0)),
                  pl.BlockSpec((H0, H1), lambda t: (0, 0)),
                  pl.BlockSpec((1, H1), lambda t: (0, 0)),
                  pl.BlockSpec((H1, H1), lambda t: (0, 0)),
                  pl.BlockSpec((1, H1), lambda t: (0, 0)),
                  pl.BlockSpec((H1, 3), lambda t: (0, 0)),
                  pl.BlockSpec((1, 3), lambda t: (0, 0))],
        out_specs=[pl.BlockSpec((TM, DOUT), lambda t: (t, 0)),
                   pl.BlockSpec((TM, 3), lambda t: (t, 0)),
                   pl.BlockSpec((TM, 3), lambda t: (t, 0))],
        out_shape=[jax.ShapeDtypeStruct((R, DOUT), jnp.float32),
                   jax.ShapeDtypeStruct((R, 3), jnp.float32),
                   jax.ShapeDtypeStruct((R, 3), jnp.float32)],
        compiler_params=pltpu.CompilerParams(
            dimension_semantics=("parallel",)),
    )


def kernel(points, batch_size, sa0_w0, sa0_b0, sa0_w1, sa0_b1,
           sa1_w0, sa1_b0, sa1_w1, sa1_b1,
           vote_w0, vote_b0, vote_reg_w, vote_reg_b,
           sa3_w0, sa3_b0, sa3_w1, sa3_b1):
    B = 4
    N = points.shape[0] // B
    xyz = points[:, 1:4].reshape(B, N, 3)

    c0 = xyz[:, :1024]
    table0 = jnp.pad(points[:, 1:5], ((0, 0), (0, 12)))
    idx0 = _make_topk(B, 1024, N, 256, 16, 0.8 * 0.8)(c0, xyz)
    g0 = _sc_gather(table0, idx0.reshape(-1), 16)
    table1 = _make_mlp(B * 1024, 1024, 16, 1, 16, 32, 16, DOUT=48)(
        g0, c0.reshape(-1, 3),
        sa0_w0, sa0_b0.reshape(1, -1), sa0_w1, sa0_b1.reshape(1, -1))

    c1 = c0[:, :256]
    idx1 = _make_topk(B, 256, 1024, 256, 16, 1.6 * 1.6)(c1, c0)
    g1 = _sc_gather(table1, idx1.reshape(-1), 48)
    table3, ctr_off, vote_xyz = _make_mlp_vote(
        B * 256, 512, 48, 32, 64, 128, 16, DOUT=144)(
        g1, c1.reshape(-1, 3),
        sa1_w0, sa1_b0.reshape(1, -1), sa1_w1, sa1_b1.reshape(1, -1),
        vote_w0, vote_b0.reshape(1, -1), vote_reg_w, vote_reg_b.reshape(1, -1))

    idx3 = _make_topk(B, 256, 256, 256, 16, 4.8 * 4.8)(
        vote_xyz.reshape(B, 256, 3), c1)
    g3 = _sc_gather(table3, idx3.reshape(-1), 144)
    f3 = _make_mlp(B * 256, 512, 144, 128, 256, 256, 16)(
        g3, vote_xyz,
        sa3_w0, sa3_b0.reshape(1, -1), sa3_w1, sa3_b1.reshape(1, -1))

    bz = (jnp.asarray(batch_size, jnp.int32) - jnp.int32(B)).astype(jnp.float32)
    ctr_batch = points[:, 0].reshape(B, N)[:, :256].reshape(-1) + bz
    centers = jnp.concatenate([ctr_batch[:, None], vote_xyz], axis=1)
    centers_origin = jnp.concatenate([ctr_batch[:, None], c1.reshape(-1, 3)],
                                     axis=1)
    ctr_offsets = jnp.concatenate([ctr_batch[:, None], ctr_off], axis=1)
    return f3, centers, centers_origin, ctr_offsets
